# baseline (speedup 1.0000x reference)
import functools

import jax
import jax.numpy as jnp
from jax import lax
from jax.experimental import pallas as pl
from jax.experimental.pallas import tpu as pltpu
from jax.experimental.pallas import tpu_sc as plsc

N_ROWS = 64
N_COLS = 32768
NUM_CORES = 2
NUM_SUBCORES = 16
NUM_WORKERS = NUM_CORES * NUM_SUBCORES
ROWS_PER_WORKER = N_ROWS // NUM_WORKERS
LANES = 16
CHUNKS = N_COLS // LANES

_mesh = plsc.VectorSubcoreMesh(core_axis_name="c", subcore_axis_name="s")


@functools.partial(
    pl.kernel,
    mesh=_mesh,
    out_type=jax.ShapeDtypeStruct((NUM_WORKERS, LANES), jnp.int32),
    scratch_types=[
        pltpu.VMEM((ROWS_PER_WORKER, N_COLS), jnp.float32),
        pltpu.VMEM((LANES,), jnp.int32),
    ],
)
def _argmin_sc(x_hbm, out_hbm, rows_v, res_v):
    wid = lax.axis_index("s") * NUM_CORES + lax.axis_index("c")
    base_row = wid * ROWS_PER_WORKER
    pltpu.sync_copy(x_hbm.at[pl.ds(base_row, ROWS_PER_WORKER)], rows_v)

    lane_iota = lax.iota(jnp.int32, LANES)
    res_vec = jnp.zeros((LANES,), jnp.int32)
    for r in range(ROWS_PER_WORKER):
        row_ref = rows_v.at[r]

        def chunk_body(c, carry, row_ref=row_ref):
            bv, bi = carry
            v = row_ref[pl.ds(c * LANES, LANES)]
            idx = lane_iota + c * LANES
            upd = v < bv
            return jnp.minimum(bv, v), jnp.where(upd, idx, bi)

        bv0 = jnp.full((LANES,), jnp.inf, jnp.float32)
        bi0 = jnp.zeros((LANES,), jnp.int32)
        bv, bi = lax.fori_loop(0, CHUNKS, chunk_body, (bv0, bi0))

        for shift in (8, 4, 2, 1):
            partner = lane_iota ^ shift
            pv = bv.at[partner].get(mode="promise_in_bounds", unique_indices=True)
            pi = bi.at[partner].get(mode="promise_in_bounds", unique_indices=True)
            upd = (pv < bv) | ((pv == bv) & (pi < bi))
            bv = jnp.where(upd, pv, bv)
            bi = jnp.where(upd, pi, bi)
        res_vec = jnp.where(lane_iota == r, bi, res_vec)

    res_v[...] = res_vec
    pltpu.sync_copy(res_v, out_hbm.at[wid])


def kernel(x):
    out2d = _argmin_sc(x)
    return out2d[:, :ROWS_PER_WORKER].reshape(-1)
